# baseline (speedup 1.0000x reference)
import functools

import jax
import jax.numpy as jnp
from jax import lax
from jax.experimental import pallas as pl
from jax.experimental.pallas import tpu as pltpu
from jax.experimental.pallas import tpu_sc as plsc

B = 16384
D = 32
NC = 2
NS = 16
NW = NC * NS
BPW = B // NW
CHUNK = 128
NCHUNK = BPW // CHUNK
ROWS = B // CHUNK


def _sc_gather(W_center16, W_outside16, b_center_flat, b_outside_flat, ci, oi):
    mesh = plsc.VectorSubcoreMesh(core_axis_name="c", subcore_axis_name="s")

    out_type = (
        jax.ShapeDtypeStruct((ROWS, CHUNK, D), jnp.bfloat16),
        jax.ShapeDtypeStruct((ROWS, CHUNK, D), jnp.bfloat16),
        jax.ShapeDtypeStruct((ROWS, CHUNK), jnp.float32),
        jax.ShapeDtypeStruct((ROWS, CHUNK), jnp.float32),
    )
    scratch = [
        pltpu.VMEM((NCHUNK, CHUNK), jnp.int32),
        pltpu.VMEM((NCHUNK, CHUNK), jnp.int32),
        pltpu.VMEM((NCHUNK, CHUNK, D), jnp.bfloat16),
        pltpu.VMEM((NCHUNK, CHUNK, D), jnp.bfloat16),
        pltpu.VMEM((NCHUNK, CHUNK), jnp.float32),
        pltpu.VMEM((NCHUNK, CHUNK), jnp.float32),
        pltpu.SemaphoreType.DMA,
    ]

    @functools.partial(pl.kernel, mesh=mesh, out_type=out_type,
                       scratch_types=scratch,
                       compiler_params=pltpu.CompilerParams(
                           use_tc_tiling_on_sc=False))
    def kern(wc_hbm, wo_hbm, bc_hbm, bo_hbm, ci_hbm, oi_hbm,
             ce_out, oe_out, cb_out, ob_out,
             ci_v, oi_v, ce_v, oe_v, cb_v, ob_v, sem):
        wid = lax.axis_index("s") * NC + lax.axis_index("c")
        row0 = wid * NCHUNK
        pltpu.sync_copy(ci_hbm.at[pl.ds(row0, NCHUNK)], ci_v)
        pltpu.sync_copy(oi_hbm.at[pl.ds(row0, NCHUNK)], oi_v)
        copies = []
        for j in range(NCHUNK):
            copies.append(pltpu.async_copy(wc_hbm.at[ci_v.at[j]], ce_v.at[j], sem))
            copies.append(pltpu.async_copy(wo_hbm.at[oi_v.at[j]], oe_v.at[j], sem))
            copies.append(pltpu.async_copy(bc_hbm.at[ci_v.at[j]], cb_v.at[j], sem))
            copies.append(pltpu.async_copy(bo_hbm.at[oi_v.at[j]], ob_v.at[j], sem))
        for c in copies:
            c.wait()
        pltpu.sync_copy(ce_v, ce_out.at[pl.ds(row0, NCHUNK)])
        pltpu.sync_copy(oe_v, oe_out.at[pl.ds(row0, NCHUNK)])
        pltpu.sync_copy(cb_v, cb_out.at[pl.ds(row0, NCHUNK)])
        pltpu.sync_copy(ob_v, ob_out.at[pl.ds(row0, NCHUNK)])

    return kern(W_center16, W_outside16, b_center_flat, b_outside_flat, ci, oi)


def _loss_body(ce_ref, oe_ref, cb_ref, ob_ref, cc_ref, w_ref, out_ref):
    prod = ce_ref[...].astype(jnp.float32) * oe_ref[...].astype(jnp.float32)
    ip = jnp.sum(prod, axis=2)
    pred = ip + cb_ref[...] + ob_ref[...]
    diff = pred - jnp.log(cc_ref[...])
    loss = w_ref[...] * diff * diff
    out_ref[...] = (jnp.sum(loss) * (1.0 / B)).reshape(1, 1)


def _tc_loss(ce, oe, cb, ob, coocs, weighting):
    return pl.pallas_call(
        _loss_body,
        out_shape=jax.ShapeDtypeStruct((1, 1), jnp.float32),
    )(ce, oe, cb, ob, coocs, weighting)


def kernel(center, outside, coocs, weighting, W_center, W_outside,
           b_center, b_outside):
    ci = center.reshape(ROWS, CHUNK)
    oi = outside.reshape(ROWS, CHUNK)
    ce, oe, cb, ob = _sc_gather(
        W_center.astype(jnp.bfloat16), W_outside.astype(jnp.bfloat16),
        b_center.reshape(-1), b_outside.reshape(-1), ci, oi)
    cc2 = coocs.reshape(ROWS, CHUNK)
    w2 = weighting.reshape(ROWS, CHUNK)
    out = _tc_loss(ce, oe, cb, ob, cc2, w2)
    return out[0, 0]


# speedup vs baseline: 1.2106x; 1.2106x over previous
import functools

import jax
import jax.numpy as jnp
from jax import lax
from jax.experimental import pallas as pl
from jax.experimental.pallas import tpu as pltpu
from jax.experimental.pallas import tpu_sc as plsc

B = 16384
D = 32
NC = 2
NS = 16
NW = NC * NS
BPW = B // NW
CHUNK = 128
NCHUNK = BPW // CHUNK
ROWS = B // CHUNK


def _sc_gather(W_center, W_outside, b_center_flat, b_outside_flat, ci, oi):
    mesh = plsc.VectorSubcoreMesh(core_axis_name="c", subcore_axis_name="s")

    out_type = (
        jax.ShapeDtypeStruct((ROWS, CHUNK, D), jnp.float32),
        jax.ShapeDtypeStruct((ROWS, CHUNK, D), jnp.float32),
        jax.ShapeDtypeStruct((ROWS, CHUNK), jnp.float32),
        jax.ShapeDtypeStruct((ROWS, CHUNK), jnp.float32),
    )
    scratch = [
        pltpu.VMEM((NCHUNK, CHUNK), jnp.int32),
        pltpu.VMEM((NCHUNK, CHUNK), jnp.int32),
        pltpu.VMEM((NCHUNK, CHUNK, D), jnp.float32),
        pltpu.VMEM((NCHUNK, CHUNK, D), jnp.float32),
        pltpu.VMEM((NCHUNK, CHUNK), jnp.float32),
        pltpu.VMEM((NCHUNK, CHUNK), jnp.float32),
        pltpu.SemaphoreType.DMA,
    ]

    @functools.partial(pl.kernel, mesh=mesh, out_type=out_type,
                       scratch_types=scratch,
                       compiler_params=pltpu.CompilerParams(
                           use_tc_tiling_on_sc=False))
    def kern(wc_hbm, wo_hbm, bc_hbm, bo_hbm, ci_hbm, oi_hbm,
             ce_out, oe_out, cb_out, ob_out,
             ci_v, oi_v, ce_v, oe_v, cb_v, ob_v, sem):
        wid = lax.axis_index("s") * NC + lax.axis_index("c")
        row0 = wid * NCHUNK
        pltpu.sync_copy(ci_hbm.at[pl.ds(row0, NCHUNK)], ci_v)
        pltpu.sync_copy(oi_hbm.at[pl.ds(row0, NCHUNK)], oi_v)
        copies = []
        for j in range(NCHUNK):
            copies.append(pltpu.async_copy(wc_hbm.at[ci_v.at[j]], ce_v.at[j], sem))
            copies.append(pltpu.async_copy(wo_hbm.at[oi_v.at[j]], oe_v.at[j], sem))
            copies.append(pltpu.async_copy(bc_hbm.at[ci_v.at[j]], cb_v.at[j], sem))
            copies.append(pltpu.async_copy(bo_hbm.at[oi_v.at[j]], ob_v.at[j], sem))
        for c in copies:
            c.wait()
        pltpu.sync_copy(ce_v, ce_out.at[pl.ds(row0, NCHUNK)])
        pltpu.sync_copy(oe_v, oe_out.at[pl.ds(row0, NCHUNK)])
        pltpu.sync_copy(cb_v, cb_out.at[pl.ds(row0, NCHUNK)])
        pltpu.sync_copy(ob_v, ob_out.at[pl.ds(row0, NCHUNK)])

    return kern(W_center, W_outside, b_center_flat, b_outside_flat, ci, oi)


def _loss_body(ce_ref, oe_ref, cb_ref, ob_ref, cc_ref, w_ref, out_ref):
    prod = ce_ref[...] * oe_ref[...]
    ip = jnp.sum(prod, axis=2)
    pred = ip + cb_ref[...] + ob_ref[...]
    diff = pred - jnp.log(cc_ref[...])
    loss = w_ref[...] * diff * diff
    out_ref[...] = (jnp.sum(loss) * (1.0 / B)).reshape(1, 1)


def _tc_loss(ce, oe, cb, ob, coocs, weighting):
    return pl.pallas_call(
        _loss_body,
        out_shape=jax.ShapeDtypeStruct((1, 1), jnp.float32),
    )(ce, oe, cb, ob, coocs, weighting)


def kernel(center, outside, coocs, weighting, W_center, W_outside,
           b_center, b_outside):
    ci = center.reshape(ROWS, CHUNK)
    oi = outside.reshape(ROWS, CHUNK)
    ce, oe, cb, ob = _sc_gather(
        W_center, W_outside,
        b_center.reshape(-1), b_outside.reshape(-1), ci, oi)
    cc2 = coocs.reshape(ROWS, CHUNK)
    w2 = weighting.reshape(ROWS, CHUNK)
    out = _tc_loss(ce, oe, cb, ob, cc2, w2)
    return out[0, 0]
